# initial kernel scaffold (unmeasured)
import jax
import jax.numpy as jnp
from jax import lax
from jax.experimental import pallas as pl
from jax.experimental.pallas import tpu as pltpu

N_DEV = 4
M = 4096
K = 1024
N = 8192
M_BLK = M // N_DEV
HALF = N // 2
WT = 1024
T = HALF // WT
N_HOP = N_DEV - 1

_sem_signal = getattr(pl, "semaphore_signal", None) or pltpu.semaphore_signal
_sem_wait = getattr(pl, "semaphore_wait", None) or pltpu.semaphore_wait
_DevId = getattr(pl, "DeviceIdType", None) or pltpu.DeviceIdType


def kernel(x, w_mat, scale_x, scale_w):
    def body(x_hbm, w_hbm, sx_ref, sw_ref, out_hbm,
             x8, w8, stage_a, stage_b, comm_cw, comm_ccw,
             send_cw, recv_cw, send_ccw, recv_ccw, copy_sems):
        d = lax.axis_index("i")
        right = jnp.mod(d + 1, N_DEV)
        left = jnp.mod(d - 1, N_DEV)

        for j in range(M // M_BLK):
            cp = pltpu.make_async_copy(
                x_hbm.at[pl.ds(j * M_BLK, M_BLK), :], stage_a, copy_sems.at[0])
            cp.start()
            cp.wait()
            x8[pl.ds(j * M_BLK, M_BLK), :] = stage_a[...].astype(jnp.float8_e4m3fn)
        for j in range(N // WT):
            cp = pltpu.make_async_copy(
                w_hbm.at[:, pl.ds(j * WT, WT)], stage_b, copy_sems.at[1])
            cp.start()
            cp.wait()
            w8[:, pl.ds(j * WT, WT)] = stage_b[...].astype(jnp.float8_e5m2)

        barrier = pltpu.get_barrier_semaphore()
        for nbr in (left, right):
            _sem_signal(barrier, inc=1, device_id=(nbr,),
                        device_id_type=_DevId.MESH)
        _sem_wait(barrier, 2)

        scale = sx_ref[0] * sw_ref[0]

        def partial(c, col0):
            xs = x8[pl.ds(c * M_BLK, M_BLK), :]
            ws = w8[:, pl.ds(col0, WT)]
            return jnp.dot(xs, ws, preferred_element_type=jnp.float32)

        for t in range(T):
            col_cw = t * WT
            col_ccw = HALF + t * WT

            comm_cw[0] = partial(jnp.mod(d - 1, N_DEV), col_cw).astype(jnp.bfloat16)
            comm_ccw[0] = partial(jnp.mod(d + 1, N_DEV), col_ccw).astype(jnp.bfloat16)

            for h in range(N_HOP):
                s, r = h, (h + 1) % 3
                rd_cw = pltpu.make_async_remote_copy(
                    src_ref=comm_cw.at[s], dst_ref=comm_cw.at[r],
                    send_sem=send_cw.at[t, h], recv_sem=recv_cw.at[t, h],
                    device_id=(right,), device_id_type=_DevId.MESH)
                rd_ccw = pltpu.make_async_remote_copy(
                    src_ref=comm_ccw.at[s], dst_ref=comm_ccw.at[r],
                    send_sem=send_ccw.at[t, h], recv_sem=recv_ccw.at[t, h],
                    device_id=(left,), device_id_type=_DevId.MESH)
                rd_cw.start()
                rd_ccw.start()
                rd_cw.wait()
                rd_ccw.wait()

                p_cw = (partial(jnp.mod(d - 2 - h, N_DEV), col_cw)
                        + comm_cw[r].astype(jnp.float32))
                p_ccw = (partial(jnp.mod(d + 2 + h, N_DEV), col_ccw)
                         + comm_ccw[r].astype(jnp.float32))
                if h < N_HOP - 1:
                    comm_cw[r] = p_cw.astype(jnp.bfloat16)
                    comm_ccw[r] = p_ccw.astype(jnp.bfloat16)
                else:
                    stage_a[...] = jnp.maximum(p_cw * scale, 0.0)
                    cp = pltpu.make_async_copy(
                        stage_a, out_hbm.at[:, pl.ds(col_cw, WT)], copy_sems.at[0])
                    cp.start()
                    cp.wait()
                    stage_b[...] = jnp.maximum(p_ccw * scale, 0.0)
                    cp = pltpu.make_async_copy(
                        stage_b, out_hbm.at[:, pl.ds(col_ccw, WT)], copy_sems.at[1])
                    cp.start()
                    cp.wait()

    return pl.pallas_call(
        body,
        out_shape=jax.ShapeDtypeStruct((M_BLK, N), jnp.float32),
        in_specs=[
            pl.BlockSpec(memory_space=pltpu.ANY),
            pl.BlockSpec(memory_space=pltpu.ANY),
            pl.BlockSpec(memory_space=pltpu.SMEM),
            pl.BlockSpec(memory_space=pltpu.SMEM),
        ],
        out_specs=pl.BlockSpec(memory_space=pltpu.ANY),
        scratch_shapes=[
            pltpu.VMEM((M, K), jnp.float8_e4m3fn),
            pltpu.VMEM((K, N), jnp.float8_e5m2),
            pltpu.VMEM((M_BLK, WT), jnp.float32),
            pltpu.VMEM((M_BLK, WT), jnp.float32),
            pltpu.VMEM((3, M_BLK, WT), jnp.bfloat16),
            pltpu.VMEM((3, M_BLK, WT), jnp.bfloat16),
            pltpu.SemaphoreType.DMA((T, N_HOP)),
            pltpu.SemaphoreType.DMA((T, N_HOP)),
            pltpu.SemaphoreType.DMA((T, N_HOP)),
            pltpu.SemaphoreType.DMA((T, N_HOP)),
            pltpu.SemaphoreType.DMA((2,)),
        ],
        compiler_params=pltpu.CompilerParams(collective_id=0),
    )(x, w_mat, scale_x, scale_w)


# baseline (device time: 402657 ns/iter reference)
import jax
import jax.numpy as jnp
from jax import lax
from jax.experimental import pallas as pl
from jax.experimental.pallas import tpu as pltpu

N_DEV = 4
M = 4096
K = 1024
N = 8192
M_BLK = M // N_DEV
HALF = N // 2
WT = 1024
T = HALF // WT
N_HOP = N_DEV - 1

_sem_signal = getattr(pl, "semaphore_signal", None) or pltpu.semaphore_signal
_sem_wait = getattr(pl, "semaphore_wait", None) or pltpu.semaphore_wait
_DevId = getattr(pl, "DeviceIdType", None) or pltpu.DeviceIdType


def kernel(x, w_mat, scale_x, scale_w):
    def body(x_hbm, w_hbm, sx_ref, sw_ref, out_hbm,
             x8, w8, stage_a, stage_b, comm_cw, comm_ccw,
             send_cw, recv_cw, send_ccw, recv_ccw, copy_sems):
        d = lax.axis_index("i")
        right = jnp.mod(d + 1, N_DEV)
        left = jnp.mod(d - 1, N_DEV)

        for j in range(M // M_BLK):
            cp = pltpu.make_async_copy(
                x_hbm.at[pl.ds(j * M_BLK, M_BLK), :], stage_a, copy_sems.at[0])
            cp.start()
            cp.wait()
            x8[pl.ds(j * M_BLK, M_BLK), :] = stage_a[...].astype(jnp.float8_e4m3fn)
        for j in range(N // WT):
            cp = pltpu.make_async_copy(
                w_hbm.at[:, pl.ds(j * WT, WT)], stage_b, copy_sems.at[1])
            cp.start()
            cp.wait()
            w8[:, pl.ds(j * WT, WT)] = stage_b[...].astype(jnp.float8_e5m2)

        barrier = pltpu.get_barrier_semaphore()
        for nbr in (left, right):
            _sem_signal(barrier, inc=1, device_id=(nbr,),
                        device_id_type=_DevId.MESH)
        _sem_wait(barrier, 2)

        scale = sx_ref[0] * sw_ref[0]

        def partial(c, col0):
            xs = x8[pl.ds(c * M_BLK, M_BLK), :]
            ws = w8[:, pl.ds(col0, WT)]
            return jnp.dot(xs, ws, preferred_element_type=jnp.float32)

        for t in range(T):
            col_cw = t * WT
            col_ccw = HALF + t * WT

            comm_cw[0] = partial(jnp.mod(d - 1, N_DEV), col_cw).astype(jnp.bfloat16)
            comm_ccw[0] = partial(jnp.mod(d + 1, N_DEV), col_ccw).astype(jnp.bfloat16)

            for h in range(N_HOP):
                s, r = h, (h + 1) % 3
                rd_cw = pltpu.make_async_remote_copy(
                    src_ref=comm_cw.at[s], dst_ref=comm_cw.at[r],
                    send_sem=send_cw.at[t, h], recv_sem=recv_cw.at[t, h],
                    device_id=(right,), device_id_type=_DevId.MESH)
                rd_ccw = pltpu.make_async_remote_copy(
                    src_ref=comm_ccw.at[s], dst_ref=comm_ccw.at[r],
                    send_sem=send_ccw.at[t, h], recv_sem=recv_ccw.at[t, h],
                    device_id=(left,), device_id_type=_DevId.MESH)
                rd_cw.start()
                rd_ccw.start()
                rd_cw.wait()
                rd_ccw.wait()

                p_cw = (partial(jnp.mod(d - 2 - h, N_DEV), col_cw)
                        + comm_cw[r].astype(jnp.float32))
                p_ccw = (partial(jnp.mod(d + 2 + h, N_DEV), col_ccw)
                         + comm_ccw[r].astype(jnp.float32))
                if h < N_HOP - 1:
                    comm_cw[r] = p_cw.astype(jnp.bfloat16)
                    comm_ccw[r] = p_ccw.astype(jnp.bfloat16)
                else:
                    stage_a[...] = jnp.maximum(p_cw * scale, 0.0)
                    cp = pltpu.make_async_copy(
                        stage_a, out_hbm.at[:, pl.ds(col_cw, WT)], copy_sems.at[0])
                    cp.start()
                    cp.wait()
                    stage_b[...] = jnp.maximum(p_ccw * scale, 0.0)
                    cp = pltpu.make_async_copy(
                        stage_b, out_hbm.at[:, pl.ds(col_ccw, WT)], copy_sems.at[1])
                    cp.start()
                    cp.wait()

    return pl.pallas_call(
        body,
        out_shape=jax.ShapeDtypeStruct((M_BLK, N), jnp.float32),
        in_specs=[
            pl.BlockSpec(memory_space=pl.ANY),
            pl.BlockSpec(memory_space=pl.ANY),
            pl.BlockSpec(memory_space=pltpu.SMEM),
            pl.BlockSpec(memory_space=pltpu.SMEM),
        ],
        out_specs=pl.BlockSpec(memory_space=pl.ANY),
        scratch_shapes=[
            pltpu.VMEM((M, K), jnp.float8_e4m3fn),
            pltpu.VMEM((K, N), jnp.float8_e5m2),
            pltpu.VMEM((M_BLK, WT), jnp.float32),
            pltpu.VMEM((M_BLK, WT), jnp.float32),
            pltpu.VMEM((3, M_BLK, WT), jnp.bfloat16),
            pltpu.VMEM((3, M_BLK, WT), jnp.bfloat16),
            pltpu.SemaphoreType.DMA((T, N_HOP)),
            pltpu.SemaphoreType.DMA((T, N_HOP)),
            pltpu.SemaphoreType.DMA((T, N_HOP)),
            pltpu.SemaphoreType.DMA((T, N_HOP)),
            pltpu.SemaphoreType.DMA((2,)),
        ],
        compiler_params=pltpu.CompilerParams(
            collective_id=0,
            vmem_limit_bytes=60 * 1024 * 1024,
        ),
    )(x, w_mat, scale_x, scale_w)


# device time: 331900 ns/iter; 1.2132x vs baseline; 1.2132x over previous
import jax
import jax.numpy as jnp
from jax import lax
from jax.experimental import pallas as pl
from jax.experimental.pallas import tpu as pltpu

N_DEV = 4
M = 4096
K = 1024
N = 8192
M_BLK = M // N_DEV
HALF = N // 2
WT = 1024
T = HALF // WT
N_HOP = N_DEV - 1

_sem_signal = getattr(pl, "semaphore_signal", None) or pltpu.semaphore_signal
_sem_wait = getattr(pl, "semaphore_wait", None) or pltpu.semaphore_wait
_DevId = getattr(pl, "DeviceIdType", None) or pltpu.DeviceIdType


def kernel(x, w_mat, scale_x, scale_w):
    def body(x_hbm, w_hbm, sx_ref, sw_ref, out_hbm,
             x8, w8, stage_a, stage_b, comm_cw, comm_ccw,
             send_cw, recv_cw, send_ccw, recv_ccw, copy_sems):
        d = lax.axis_index("i")
        right = jnp.mod(d + 1, N_DEV)
        left = jnp.mod(d - 1, N_DEV)
        stages = (stage_a, stage_b)

        def x_src(j):
            return x_hbm.at[pl.ds(j * M_BLK, M_BLK), :]

        def w_src(j):
            return w_hbm.at[:, pl.ds(j * WT, WT)]

        def x_sink(j, buf):
            x8[pl.ds(j * M_BLK, M_BLK), :] = buf[...].astype(jnp.float8_e4m3fn)

        def w_sink(j, buf):
            w8[:, pl.ds(j * WT, WT)] = buf[...].astype(jnp.float8_e5m2)

        plan = ([(x_src, x_sink, j) for j in range(M // M_BLK)]
                + [(w_src, w_sink, j) for j in range(N // WT)])
        pending = None
        for k in range(len(plan) + 1):
            if k < len(plan):
                src, sink, j = plan[k]
                cp = pltpu.make_async_copy(src(j), stages[k % 2],
                                           copy_sems.at[k % 2])
                cp.start()
            if pending is not None:
                pcp, psink, pj, pbuf = pending
                pcp.wait()
                psink(pj, stages[pbuf])
            pending = (cp, sink, j, k % 2) if k < len(plan) else None

        barrier = pltpu.get_barrier_semaphore()
        for nbr in (left, right):
            _sem_signal(barrier, inc=1, device_id=(nbr,),
                        device_id_type=_DevId.MESH)
        _sem_wait(barrier, 2)

        scale = sx_ref[0] * sw_ref[0]

        def partial(c, col0):
            xs = x8[pl.ds(c * M_BLK, M_BLK), :]
            ws = w8[:, pl.ds(col0, WT)]
            return jnp.dot(xs, ws, preferred_element_type=jnp.float32)

        def make_rdma(t, h):
            b, s, r = t % 2, h, (h + 1) % 3
            rd_cw = pltpu.make_async_remote_copy(
                src_ref=comm_cw.at[b, s], dst_ref=comm_cw.at[b, r],
                send_sem=send_cw.at[t, h], recv_sem=recv_cw.at[t, h],
                device_id=(right,), device_id_type=_DevId.MESH)
            rd_ccw = pltpu.make_async_remote_copy(
                src_ref=comm_ccw.at[b, s], dst_ref=comm_ccw.at[b, r],
                send_sem=send_ccw.at[t, h], recv_sem=recv_ccw.at[t, h],
                device_id=(left,), device_id_type=_DevId.MESH)
            return rd_cw, rd_ccw

        inflight = {}
        for p in range(0, T, 2):
            pair = (p, p + 1)
            for t in pair:
                b = t % 2
                comm_cw[b, 0] = partial(jnp.mod(d - 1, N_DEV),
                                        t * WT).astype(jnp.bfloat16)
                comm_ccw[b, 0] = partial(jnp.mod(d + 1, N_DEV),
                                         HALF + t * WT).astype(jnp.bfloat16)
                rds = make_rdma(t, 0)
                rds[0].start()
                rds[1].start()
                inflight[t] = rds
            for h in range(N_HOP):
                for t in pair:
                    b, r = t % 2, (h + 1) % 3
                    col_cw, col_ccw = t * WT, HALF + t * WT
                    rd_cw, rd_ccw = inflight.pop(t)
                    rd_cw.wait()
                    rd_ccw.wait()
                    p_cw = (partial(jnp.mod(d - 2 - h, N_DEV), col_cw)
                            + comm_cw[b, r].astype(jnp.float32))
                    p_ccw = (partial(jnp.mod(d + 2 + h, N_DEV), col_ccw)
                             + comm_ccw[b, r].astype(jnp.float32))
                    if h < N_HOP - 1:
                        comm_cw[b, r] = p_cw.astype(jnp.bfloat16)
                        comm_ccw[b, r] = p_ccw.astype(jnp.bfloat16)
                        rds = make_rdma(t, h + 1)
                        rds[0].start()
                        rds[1].start()
                        inflight[t] = rds
                    else:
                        stage_a[...] = jnp.maximum(p_cw * scale, 0.0)
                        cp = pltpu.make_async_copy(
                            stage_a, out_hbm.at[:, pl.ds(col_cw, WT)],
                            copy_sems.at[0])
                        cp.start()
                        stage_b[...] = jnp.maximum(p_ccw * scale, 0.0)
                        cp2 = pltpu.make_async_copy(
                            stage_b, out_hbm.at[:, pl.ds(col_ccw, WT)],
                            copy_sems.at[1])
                        cp2.start()
                        cp.wait()
                        cp2.wait()

    return pl.pallas_call(
        body,
        out_shape=jax.ShapeDtypeStruct((M_BLK, N), jnp.float32),
        in_specs=[
            pl.BlockSpec(memory_space=pl.ANY),
            pl.BlockSpec(memory_space=pl.ANY),
            pl.BlockSpec(memory_space=pltpu.SMEM),
            pl.BlockSpec(memory_space=pltpu.SMEM),
        ],
        out_specs=pl.BlockSpec(memory_space=pl.ANY),
        scratch_shapes=[
            pltpu.VMEM((M, K), jnp.float8_e4m3fn),
            pltpu.VMEM((K, N), jnp.float8_e5m2),
            pltpu.VMEM((M_BLK, WT), jnp.float32),
            pltpu.VMEM((M_BLK, WT), jnp.float32),
            pltpu.VMEM((2, 3, M_BLK, WT), jnp.bfloat16),
            pltpu.VMEM((2, 3, M_BLK, WT), jnp.bfloat16),
            pltpu.SemaphoreType.DMA((T, N_HOP)),
            pltpu.SemaphoreType.DMA((T, N_HOP)),
            pltpu.SemaphoreType.DMA((T, N_HOP)),
            pltpu.SemaphoreType.DMA((T, N_HOP)),
            pltpu.SemaphoreType.DMA((2,)),
        ],
        compiler_params=pltpu.CompilerParams(
            collective_id=0,
            vmem_limit_bytes=60 * 1024 * 1024,
        ),
    )(x, w_mat, scale_x, scale_w)


# device time: 317379 ns/iter; 1.2687x vs baseline; 1.0458x over previous
import jax
import jax.numpy as jnp
from jax import lax
from jax.experimental import pallas as pl
from jax.experimental.pallas import tpu as pltpu

N_DEV = 4
M = 4096
K = 1024
N = 8192
M_BLK = M // N_DEV
HALF = N // 2
WT = 1024
T = HALF // WT
N_HOP = N_DEV - 1

_sem_signal = getattr(pl, "semaphore_signal", None) or pltpu.semaphore_signal
_sem_wait = getattr(pl, "semaphore_wait", None) or pltpu.semaphore_wait
_DevId = getattr(pl, "DeviceIdType", None) or pltpu.DeviceIdType


def kernel(x, w_mat, scale_x, scale_w):
    def body(x_hbm, w_hbm, sx_ref, sw_ref, out_hbm,
             x8, w8, stage_a, stage_b, comm_cw, comm_ccw,
             send_cw, recv_cw, send_ccw, recv_ccw, copy_sems):
        d = lax.axis_index("i")
        right = jnp.mod(d + 1, N_DEV)
        left = jnp.mod(d - 1, N_DEV)
        stages = (stage_a, stage_b)

        def x_src(j):
            return x_hbm.at[pl.ds(j * M_BLK, M_BLK), :]

        def w_src(j):
            return w_hbm.at[:, pl.ds(j * WT, WT)]

        def x_sink(j, buf):
            x8[pl.ds(j * M_BLK, M_BLK), :] = buf[...].astype(jnp.float8_e4m3fn)

        def w_sink(j, buf):
            w8[:, pl.ds(j * WT, WT)] = buf[...].astype(jnp.float8_e5m2)

        def run_staging(plan):
            pending = None
            for k in range(len(plan) + 1):
                if k < len(plan):
                    src, sink, j = plan[k]
                    cp = pltpu.make_async_copy(src(j), stages[k % 2],
                                               copy_sems.at[k % 2])
                    cp.start()
                if pending is not None:
                    pcp, psink, pj, pbuf = pending
                    pcp.wait()
                    psink(pj, stages[pbuf])
                pending = (cp, sink, j, k % 2) if k < len(plan) else None

        run_staging([(x_src, x_sink, j) for j in range(M // M_BLK)]
                    + [(w_src, w_sink, j) for j in (0, 1, 4, 5)])

        barrier = pltpu.get_barrier_semaphore()
        for nbr in (left, right):
            _sem_signal(barrier, inc=1, device_id=(nbr,),
                        device_id_type=_DevId.MESH)
        _sem_wait(barrier, 2)

        scale = sx_ref[0] * sw_ref[0]

        def partial(c, col0):
            xs = x8[pl.ds(c * M_BLK, M_BLK), :]
            ws = w8[:, pl.ds(col0, WT)]
            return jnp.dot(xs, ws, preferred_element_type=jnp.float32)

        def make_rdma(t, h):
            b, s, r = t % 2, h, (h + 1) % 3
            rd_cw = pltpu.make_async_remote_copy(
                src_ref=comm_cw.at[b, s], dst_ref=comm_cw.at[b, r],
                send_sem=send_cw.at[t, h], recv_sem=recv_cw.at[t, h],
                device_id=(right,), device_id_type=_DevId.MESH)
            rd_ccw = pltpu.make_async_remote_copy(
                src_ref=comm_ccw.at[b, s], dst_ref=comm_ccw.at[b, r],
                send_sem=send_ccw.at[t, h], recv_sem=recv_ccw.at[t, h],
                device_id=(left,), device_id_type=_DevId.MESH)
            return rd_cw, rd_ccw

        def init_tile(t):
            b = t % 2
            comm_cw[b, 0] = partial(jnp.mod(d - 1, N_DEV),
                                    t * WT).astype(jnp.bfloat16)
            comm_ccw[b, 0] = partial(jnp.mod(d + 1, N_DEV),
                                     HALF + t * WT).astype(jnp.bfloat16)
            rds = make_rdma(t, 0)
            rds[0].start()
            rds[1].start()
            return rds

        queue = [(t, 0, init_tile(t)) for t in (0, 1)]
        next_tile = 2

        run_staging([(w_src, w_sink, j) for j in (2, 3, 6, 7)])

        while queue:
            t, h, (rd_cw, rd_ccw) = queue.pop(0)
            b, r = t % 2, (h + 1) % 3
            col_cw, col_ccw = t * WT, HALF + t * WT
            rd_cw.wait()
            rd_ccw.wait()
            if h < N_HOP - 1:
                comm_cw[b, r] = (partial(jnp.mod(d - 2 - h, N_DEV), col_cw)
                                 + comm_cw[b, r].astype(jnp.float32)
                                 ).astype(jnp.bfloat16)
                comm_ccw[b, r] = (partial(jnp.mod(d + 2 + h, N_DEV), col_ccw)
                                  + comm_ccw[b, r].astype(jnp.float32)
                                  ).astype(jnp.bfloat16)
                rds = make_rdma(t, h + 1)
                rds[0].start()
                rds[1].start()
                queue.append((t, h + 1, rds))
            else:
                p_cw = (partial(jnp.mod(d - 2 - h, N_DEV), col_cw)
                        + comm_cw[b, r].astype(jnp.float32))
                p_ccw = (partial(jnp.mod(d + 2 + h, N_DEV), col_ccw)
                         + comm_ccw[b, r].astype(jnp.float32))
                stage_a[...] = jnp.maximum(p_cw * scale, 0.0)
                cp = pltpu.make_async_copy(
                    stage_a, out_hbm.at[:, pl.ds(col_cw, WT)],
                    copy_sems.at[0])
                cp.start()
                stage_b[...] = jnp.maximum(p_ccw * scale, 0.0)
                cp2 = pltpu.make_async_copy(
                    stage_b, out_hbm.at[:, pl.ds(col_ccw, WT)],
                    copy_sems.at[1])
                cp2.start()
                if next_tile < T:
                    queue.append((next_tile, 0, init_tile(next_tile)))
                    next_tile += 1
                cp.wait()
                cp2.wait()

    return pl.pallas_call(
        body,
        out_shape=jax.ShapeDtypeStruct((M_BLK, N), jnp.float32),
        in_specs=[
            pl.BlockSpec(memory_space=pl.ANY),
            pl.BlockSpec(memory_space=pl.ANY),
            pl.BlockSpec(memory_space=pltpu.SMEM),
            pl.BlockSpec(memory_space=pltpu.SMEM),
        ],
        out_specs=pl.BlockSpec(memory_space=pl.ANY),
        scratch_shapes=[
            pltpu.VMEM((M, K), jnp.float8_e4m3fn),
            pltpu.VMEM((K, N), jnp.float8_e5m2),
            pltpu.VMEM((M_BLK, WT), jnp.float32),
            pltpu.VMEM((M_BLK, WT), jnp.float32),
            pltpu.VMEM((2, 3, M_BLK, WT), jnp.bfloat16),
            pltpu.VMEM((2, 3, M_BLK, WT), jnp.bfloat16),
            pltpu.SemaphoreType.DMA((T, N_HOP)),
            pltpu.SemaphoreType.DMA((T, N_HOP)),
            pltpu.SemaphoreType.DMA((T, N_HOP)),
            pltpu.SemaphoreType.DMA((T, N_HOP)),
            pltpu.SemaphoreType.DMA((2,)),
        ],
        compiler_params=pltpu.CompilerParams(
            collective_id=0,
            vmem_limit_bytes=60 * 1024 * 1024,
        ),
    )(x, w_mat, scale_x, scale_w)


# device time: 312209 ns/iter; 1.2897x vs baseline; 1.0166x over previous
import jax
import jax.numpy as jnp
from jax import lax
from jax.experimental import pallas as pl
from jax.experimental.pallas import tpu as pltpu

N_DEV = 4
M = 4096
K = 1024
N = 8192
M_BLK = M // N_DEV
HALF = N // 2
WT = 1024
T = HALF // WT
N_HOP = N_DEV - 1

_sem_signal = getattr(pl, "semaphore_signal", None) or pltpu.semaphore_signal
_sem_wait = getattr(pl, "semaphore_wait", None) or pltpu.semaphore_wait
_DevId = getattr(pl, "DeviceIdType", None) or pltpu.DeviceIdType


def kernel(x, w_mat, scale_x, scale_w):
    def body(x_hbm, w_hbm, sx_ref, sw_ref, out_hbm,
             x8, w8, stage_a, stage_b, comm_cw, comm_ccw,
             send_cw, recv_cw, send_ccw, recv_ccw, copy_sems):
        d = lax.axis_index("i")
        right = jnp.mod(d + 1, N_DEV)
        left = jnp.mod(d - 1, N_DEV)
        stages = (stage_a, stage_b)

        def x_src(c):
            return x_hbm.at[pl.ds(c * M_BLK, M_BLK), :]

        def w_src(j):
            return w_hbm.at[:, pl.ds(j * WT, WT)]

        def x_sink(c, buf):
            x8[pl.ds(c * M_BLK, M_BLK), :] = buf[...].astype(jnp.float8_e4m3fn)

        def w_sink(j, buf):
            w8[:, pl.ds(j * WT, WT)] = buf[...].astype(jnp.float8_e5m2)

        def run_staging(plan):
            pending = None
            for k in range(len(plan) + 1):
                if k < len(plan):
                    src, sink, j = plan[k]
                    cp = pltpu.make_async_copy(src(j), stages[k % 2],
                                               copy_sems.at[k % 2])
                    cp.start()
                if pending is not None:
                    pcp, psink, pj, pbuf = pending
                    pcp.wait()
                    psink(pj, stages[pbuf])
                pending = (cp, sink, j, k % 2) if k < len(plan) else None

        run_staging([(x_src, x_sink, jnp.mod(d - 1, N_DEV)),
                     (x_src, x_sink, jnp.mod(d + 1, N_DEV)),
                     (w_src, w_sink, 0), (w_src, w_sink, 4)])

        barrier = pltpu.get_barrier_semaphore()
        for nbr in (left, right):
            _sem_signal(barrier, inc=1, device_id=(nbr,),
                        device_id_type=_DevId.MESH)
        _sem_wait(barrier, 2)

        scale = sx_ref[0] * sw_ref[0]

        def partial(c, col0):
            xs = x8[pl.ds(c * M_BLK, M_BLK), :]
            ws = w8[:, pl.ds(col0, WT)]
            return jnp.dot(xs, ws, preferred_element_type=jnp.float32)

        def make_rdma(t, h):
            b, s, r = t % 2, h, (h + 1) % 3
            rd_cw = pltpu.make_async_remote_copy(
                src_ref=comm_cw.at[b, s], dst_ref=comm_cw.at[b, r],
                send_sem=send_cw.at[t, h], recv_sem=recv_cw.at[t, h],
                device_id=(right,), device_id_type=_DevId.MESH)
            rd_ccw = pltpu.make_async_remote_copy(
                src_ref=comm_ccw.at[b, s], dst_ref=comm_ccw.at[b, r],
                send_sem=send_ccw.at[t, h], recv_sem=recv_ccw.at[t, h],
                device_id=(left,), device_id_type=_DevId.MESH)
            return rd_cw, rd_ccw

        def init_tile(t):
            b = t % 2
            comm_cw[b, 0] = partial(jnp.mod(d - 1, N_DEV),
                                    t * WT).astype(jnp.bfloat16)
            comm_ccw[b, 0] = partial(jnp.mod(d + 1, N_DEV),
                                     HALF + t * WT).astype(jnp.bfloat16)
            rds = make_rdma(t, 0)
            rds[0].start()
            rds[1].start()
            return rds

        queue = [(0, 0, init_tile(0))]
        run_staging([(w_src, w_sink, 1), (w_src, w_sink, 5)])
        queue.append((1, 0, init_tile(1)))
        next_tile = 2
        run_staging([(x_src, x_sink, jnp.mod(d - 2, N_DEV))])

        hooks = {
            0: [(x_src, x_sink, d), (w_src, w_sink, 2), (w_src, w_sink, 6)],
            1: [(w_src, w_sink, 3), (w_src, w_sink, 7)],
        }

        step = 0
        while queue:
            t, h, (rd_cw, rd_ccw) = queue.pop(0)
            b, r = t % 2, (h + 1) % 3
            col_cw, col_ccw = t * WT, HALF + t * WT
            rd_cw.wait()
            rd_ccw.wait()
            if h < N_HOP - 1:
                comm_cw[b, r] = (partial(jnp.mod(d - 2 - h, N_DEV), col_cw)
                                 + comm_cw[b, r].astype(jnp.float32)
                                 ).astype(jnp.bfloat16)
                comm_ccw[b, r] = (partial(jnp.mod(d + 2 + h, N_DEV), col_ccw)
                                  + comm_ccw[b, r].astype(jnp.float32)
                                  ).astype(jnp.bfloat16)
                rds = make_rdma(t, h + 1)
                rds[0].start()
                rds[1].start()
                queue.append((t, h + 1, rds))
            else:
                p_cw = (partial(jnp.mod(d - 2 - h, N_DEV), col_cw)
                        + comm_cw[b, r].astype(jnp.float32))
                p_ccw = (partial(jnp.mod(d + 2 + h, N_DEV), col_ccw)
                         + comm_ccw[b, r].astype(jnp.float32))
                stage_a[...] = jnp.maximum(p_cw * scale, 0.0)
                cp = pltpu.make_async_copy(
                    stage_a, out_hbm.at[:, pl.ds(col_cw, WT)],
                    copy_sems.at[0])
                cp.start()
                stage_b[...] = jnp.maximum(p_ccw * scale, 0.0)
                cp2 = pltpu.make_async_copy(
                    stage_b, out_hbm.at[:, pl.ds(col_ccw, WT)],
                    copy_sems.at[1])
                cp2.start()
                if next_tile < T:
                    queue.append((next_tile, 0, init_tile(next_tile)))
                    next_tile += 1
                cp.wait()
                cp2.wait()
            if step in hooks:
                run_staging(hooks.pop(step))
            step += 1

    return pl.pallas_call(
        body,
        out_shape=jax.ShapeDtypeStruct((M_BLK, N), jnp.float32),
        in_specs=[
            pl.BlockSpec(memory_space=pl.ANY),
            pl.BlockSpec(memory_space=pl.ANY),
            pl.BlockSpec(memory_space=pltpu.SMEM),
            pl.BlockSpec(memory_space=pltpu.SMEM),
        ],
        out_specs=pl.BlockSpec(memory_space=pl.ANY),
        scratch_shapes=[
            pltpu.VMEM((M, K), jnp.float8_e4m3fn),
            pltpu.VMEM((K, N), jnp.float8_e5m2),
            pltpu.VMEM((M_BLK, WT), jnp.float32),
            pltpu.VMEM((M_BLK, WT), jnp.float32),
            pltpu.VMEM((2, 3, M_BLK, WT), jnp.bfloat16),
            pltpu.VMEM((2, 3, M_BLK, WT), jnp.bfloat16),
            pltpu.SemaphoreType.DMA((T, N_HOP)),
            pltpu.SemaphoreType.DMA((T, N_HOP)),
            pltpu.SemaphoreType.DMA((T, N_HOP)),
            pltpu.SemaphoreType.DMA((T, N_HOP)),
            pltpu.SemaphoreType.DMA((2,)),
        ],
        compiler_params=pltpu.CompilerParams(
            collective_id=0,
            vmem_limit_bytes=60 * 1024 * 1024,
        ),
    )(x, w_mat, scale_x, scale_w)


# device time: 310228 ns/iter; 1.2979x vs baseline; 1.0064x over previous
import jax
import jax.numpy as jnp
from jax import lax
from jax.experimental import pallas as pl
from jax.experimental.pallas import tpu as pltpu

N_DEV = 4
M = 4096
K = 1024
N = 8192
M_BLK = M // N_DEV
HALF = N // 2
WT = 1024
T = HALF // WT
N_HOP = N_DEV - 1

_sem_signal = getattr(pl, "semaphore_signal", None) or pltpu.semaphore_signal
_sem_wait = getattr(pl, "semaphore_wait", None) or pltpu.semaphore_wait
_DevId = getattr(pl, "DeviceIdType", None) or pltpu.DeviceIdType


def kernel(x, w_mat, scale_x, scale_w):
    def body(x_hbm, w_hbm, sx_ref, sw_ref, out_hbm,
             x8, w8, stage_refs, comm_cw, comm_ccw,
             send_cw, recv_cw, send_ccw, recv_ccw, copy_sems):
        d = lax.axis_index("i")
        right = jnp.mod(d + 1, N_DEV)
        left = jnp.mod(d - 1, N_DEV)
        n_stage = 4
        stages = [stage_refs.at[i] for i in range(n_stage)]
        stage_a, stage_b = stages[0], stages[1]

        barrier = pltpu.get_barrier_semaphore()
        for nbr in (left, right):
            _sem_signal(barrier, inc=1, device_id=(nbr,),
                        device_id_type=_DevId.MESH)

        def x_src(c):
            return x_hbm.at[pl.ds(c * M_BLK, M_BLK), :]

        def w_src(j):
            return w_hbm.at[:, pl.ds(j * WT, WT)]

        def x_sink(c, buf):
            x8[pl.ds(c * M_BLK, M_BLK), :] = buf[...].astype(jnp.float8_e4m3fn)

        def w_sink(j, buf):
            w8[:, pl.ds(j * WT, WT)] = buf[...].astype(jnp.float8_e5m2)

        def run_staging(plan):
            live = []
            for k, (src, sink, j) in enumerate(plan):
                if len(live) == n_stage:
                    pcp, psink, pj, pbuf = live.pop(0)
                    pcp.wait()
                    psink(pj, stages[pbuf])
                cp = pltpu.make_async_copy(src(j), stages[k % n_stage],
                                           copy_sems.at[k % n_stage])
                cp.start()
                live.append((cp, sink, j, k % n_stage))
            for pcp, psink, pj, pbuf in live:
                pcp.wait()
                psink(pj, stages[pbuf])

        run_staging([(x_src, x_sink, jnp.mod(d - 1, N_DEV)),
                     (x_src, x_sink, jnp.mod(d + 1, N_DEV)),
                     (w_src, w_sink, 0), (w_src, w_sink, 4)])

        _sem_wait(barrier, 2)

        scale = sx_ref[0] * sw_ref[0]

        def partial(c, col0):
            xs = x8[pl.ds(c * M_BLK, M_BLK), :]
            ws = w8[:, pl.ds(col0, WT)]
            return jnp.dot(xs, ws, preferred_element_type=jnp.float32)

        def make_rdma(t, h):
            b, s, r = t % 2, h, (h + 1) % 3
            rd_cw = pltpu.make_async_remote_copy(
                src_ref=comm_cw.at[b, s], dst_ref=comm_cw.at[b, r],
                send_sem=send_cw.at[t, h], recv_sem=recv_cw.at[t, h],
                device_id=(right,), device_id_type=_DevId.MESH)
            rd_ccw = pltpu.make_async_remote_copy(
                src_ref=comm_ccw.at[b, s], dst_ref=comm_ccw.at[b, r],
                send_sem=send_ccw.at[t, h], recv_sem=recv_ccw.at[t, h],
                device_id=(left,), device_id_type=_DevId.MESH)
            return rd_cw, rd_ccw

        def init_tile(t):
            b = t % 2
            comm_cw[b, 0] = partial(jnp.mod(d - 1, N_DEV),
                                    t * WT).astype(jnp.bfloat16)
            comm_ccw[b, 0] = partial(jnp.mod(d + 1, N_DEV),
                                     HALF + t * WT).astype(jnp.bfloat16)
            rds = make_rdma(t, 0)
            rds[0].start()
            rds[1].start()
            return rds

        queue = [(0, 0, init_tile(0))]
        run_staging([(w_src, w_sink, 1), (w_src, w_sink, 5)])
        queue.append((1, 0, init_tile(1)))
        next_tile = 2
        run_staging([(x_src, x_sink, jnp.mod(d - 2, N_DEV))])

        hooks = {
            0: [(x_src, x_sink, d), (w_src, w_sink, 2), (w_src, w_sink, 6)],
            1: [(w_src, w_sink, 3), (w_src, w_sink, 7)],
        }

        step = 0
        while queue:
            t, h, (rd_cw, rd_ccw) = queue.pop(0)
            b, r = t % 2, (h + 1) % 3
            col_cw, col_ccw = t * WT, HALF + t * WT
            p_cw = partial(jnp.mod(d - 2 - h, N_DEV), col_cw)
            p_ccw = partial(jnp.mod(d + 2 + h, N_DEV), col_ccw)
            rd_cw.wait()
            rd_ccw.wait()
            if h < N_HOP - 1:
                comm_cw[b, r] = (p_cw + comm_cw[b, r].astype(jnp.float32)
                                 ).astype(jnp.bfloat16)
                comm_ccw[b, r] = (p_ccw + comm_ccw[b, r].astype(jnp.float32)
                                  ).astype(jnp.bfloat16)
                rds = make_rdma(t, h + 1)
                rds[0].start()
                rds[1].start()
                queue.append((t, h + 1, rds))
            else:
                stage_a[...] = jnp.maximum(
                    (p_cw + comm_cw[b, r].astype(jnp.float32)) * scale, 0.0)
                cp = pltpu.make_async_copy(
                    stage_a, out_hbm.at[:, pl.ds(col_cw, WT)],
                    copy_sems.at[0])
                cp.start()
                stage_b[...] = jnp.maximum(
                    (p_ccw + comm_ccw[b, r].astype(jnp.float32)) * scale, 0.0)
                cp2 = pltpu.make_async_copy(
                    stage_b, out_hbm.at[:, pl.ds(col_ccw, WT)],
                    copy_sems.at[1])
                cp2.start()
                if next_tile < T:
                    queue.append((next_tile, 0, init_tile(next_tile)))
                    next_tile += 1
                cp.wait()
                cp2.wait()
            if step in hooks:
                run_staging(hooks.pop(step))
            step += 1

    return pl.pallas_call(
        body,
        out_shape=jax.ShapeDtypeStruct((M_BLK, N), jnp.float32),
        in_specs=[
            pl.BlockSpec(memory_space=pl.ANY),
            pl.BlockSpec(memory_space=pl.ANY),
            pl.BlockSpec(memory_space=pltpu.SMEM),
            pl.BlockSpec(memory_space=pltpu.SMEM),
        ],
        out_specs=pl.BlockSpec(memory_space=pl.ANY),
        scratch_shapes=[
            pltpu.VMEM((M, K), jnp.float8_e4m3fn),
            pltpu.VMEM((K, N), jnp.float8_e5m2),
            pltpu.VMEM((4, M_BLK, WT), jnp.float32),
            pltpu.VMEM((2, 3, M_BLK, WT), jnp.bfloat16),
            pltpu.VMEM((2, 3, M_BLK, WT), jnp.bfloat16),
            pltpu.SemaphoreType.DMA((T, N_HOP)),
            pltpu.SemaphoreType.DMA((T, N_HOP)),
            pltpu.SemaphoreType.DMA((T, N_HOP)),
            pltpu.SemaphoreType.DMA((T, N_HOP)),
            pltpu.SemaphoreType.DMA((4,)),
        ],
        compiler_params=pltpu.CompilerParams(
            collective_id=0,
            vmem_limit_bytes=60 * 1024 * 1024,
        ),
    )(x, w_mat, scale_x, scale_w)


# device time: 306032 ns/iter; 1.3157x vs baseline; 1.0137x over previous
import jax
import jax.numpy as jnp
from jax import lax
from jax.experimental import pallas as pl
from jax.experimental.pallas import tpu as pltpu

N_DEV = 4
M = 4096
K = 1024
N = 8192
M_BLK = M // N_DEV
HALF = N // 2
WT = 1024
N_HOP = N_DEV - 1

WIDS = (512, 512, 1024, 1024, 512, 512)
OFFS = (0, 512, 1024, 2048, 3072, 3584)
N_TILES = len(WIDS)

_sem_signal = getattr(pl, "semaphore_signal", None) or pltpu.semaphore_signal
_sem_wait = getattr(pl, "semaphore_wait", None) or pltpu.semaphore_wait
_DevId = getattr(pl, "DeviceIdType", None) or pltpu.DeviceIdType


def kernel(x, w_mat, scale_x, scale_w):
    def body(x_hbm, w_hbm, sx_ref, sw_ref, out_hbm,
             x8, w8, stage_refs, comm_cw, comm_ccw,
             send_cw, recv_cw, send_ccw, recv_ccw, copy_sems):
        d = lax.axis_index("i")
        right = jnp.mod(d + 1, N_DEV)
        left = jnp.mod(d - 1, N_DEV)
        n_stage = 4

        barrier = pltpu.get_barrier_semaphore()
        for nbr in (left, right):
            _sem_signal(barrier, inc=1, device_id=(nbr,),
                        device_id_type=_DevId.MESH)

        def x_src(c, wt):
            return x_hbm.at[pl.ds(c * M_BLK, M_BLK), :]

        def w_src(col, wt):
            return w_hbm.at[:, pl.ds(col, wt)]

        def x_sink(c, slot, wt):
            x8[pl.ds(c * M_BLK, M_BLK), :] = (
                stage_refs[slot].astype(jnp.float8_e4m3fn))

        def w_sink(col, slot, wt):
            w8[:, pl.ds(col, wt)] = (
                stage_refs[slot, :, pl.ds(0, wt)].astype(jnp.float8_e5m2))

        def run_staging(plan):
            live = []
            for k, (src, sink, arg, wt) in enumerate(plan):
                if len(live) == n_stage:
                    pcp, psink, parg, pslot, pwt = live.pop(0)
                    pcp.wait()
                    psink(parg, pslot, pwt)
                slot = k % n_stage
                cp = pltpu.make_async_copy(
                    src(arg, wt),
                    stage_refs.at[slot, slice(None), pl.ds(0, wt)],
                    copy_sems.at[slot])
                cp.start()
                live.append((cp, sink, arg, slot, wt))
            for pcp, psink, parg, pslot, pwt in live:
                pcp.wait()
                psink(parg, pslot, pwt)

        run_staging([(x_src, x_sink, jnp.mod(d - 1, N_DEV), K),
                     (x_src, x_sink, jnp.mod(d + 1, N_DEV), K),
                     (w_src, w_sink, OFFS[0], WIDS[0]),
                     (w_src, w_sink, HALF + OFFS[0], WIDS[0])])

        _sem_wait(barrier, 2)

        scale = sx_ref[0] * sw_ref[0]

        def partial(c, col0, wt):
            xs = x8[pl.ds(c * M_BLK, M_BLK), :]
            ws = w8[:, pl.ds(col0, wt)]
            return jnp.dot(xs, ws, preferred_element_type=jnp.float32)

        def make_rdma(t, h):
            b, s, r = t % 2, h, (h + 1) % 3
            wt = WIDS[t]
            rd_cw = pltpu.make_async_remote_copy(
                src_ref=comm_cw.at[b, s, slice(None), pl.ds(0, wt)],
                dst_ref=comm_cw.at[b, r, slice(None), pl.ds(0, wt)],
                send_sem=send_cw.at[t, h], recv_sem=recv_cw.at[t, h],
                device_id=(right,), device_id_type=_DevId.MESH)
            rd_ccw = pltpu.make_async_remote_copy(
                src_ref=comm_ccw.at[b, s, slice(None), pl.ds(0, wt)],
                dst_ref=comm_ccw.at[b, r, slice(None), pl.ds(0, wt)],
                send_sem=send_ccw.at[t, h], recv_sem=recv_ccw.at[t, h],
                device_id=(left,), device_id_type=_DevId.MESH)
            return rd_cw, rd_ccw

        def init_tile(t):
            b, wt = t % 2, WIDS[t]
            comm_cw[b, 0, :, pl.ds(0, wt)] = partial(
                jnp.mod(d - 1, N_DEV), OFFS[t], wt).astype(jnp.bfloat16)
            comm_ccw[b, 0, :, pl.ds(0, wt)] = partial(
                jnp.mod(d + 1, N_DEV), HALF + OFFS[t], wt).astype(jnp.bfloat16)
            rds = make_rdma(t, 0)
            rds[0].start()
            rds[1].start()
            return rds

        queue = [(0, 0, init_tile(0))]
        run_staging([(w_src, w_sink, OFFS[1], WIDS[1]),
                     (w_src, w_sink, HALF + OFFS[1], WIDS[1])])
        queue.append((1, 0, init_tile(1)))
        next_tile = 2
        run_staging([(x_src, x_sink, jnp.mod(d - 2, N_DEV), K)])

        hooks = {
            0: [(x_src, x_sink, d, K),
                (w_src, w_sink, OFFS[2], WIDS[2]),
                (w_src, w_sink, HALF + OFFS[2], WIDS[2])],
            1: [(w_src, w_sink, OFFS[3], WIDS[3]),
                (w_src, w_sink, HALF + OFFS[3], WIDS[3])],
            2: [(w_src, w_sink, OFFS[4], WIDS[4]),
                (w_src, w_sink, HALF + OFFS[4], WIDS[4])],
            3: [(w_src, w_sink, OFFS[5], WIDS[5]),
                (w_src, w_sink, HALF + OFFS[5], WIDS[5])],
        }

        step = 0
        while queue:
            t, h, (rd_cw, rd_ccw) = queue.pop(0)
            b, r, wt = t % 2, (h + 1) % 3, WIDS[t]
            col_cw, col_ccw = OFFS[t], HALF + OFFS[t]
            p_cw = partial(jnp.mod(d - 2 - h, N_DEV), col_cw, wt)
            p_ccw = partial(jnp.mod(d + 2 + h, N_DEV), col_ccw, wt)
            rd_cw.wait()
            rd_ccw.wait()
            if h < N_HOP - 1:
                comm_cw[b, r, :, pl.ds(0, wt)] = (
                    p_cw + comm_cw[b, r, :, pl.ds(0, wt)].astype(jnp.float32)
                ).astype(jnp.bfloat16)
                comm_ccw[b, r, :, pl.ds(0, wt)] = (
                    p_ccw + comm_ccw[b, r, :, pl.ds(0, wt)].astype(jnp.float32)
                ).astype(jnp.bfloat16)
                rds = make_rdma(t, h + 1)
                rds[0].start()
                rds[1].start()
                queue.append((t, h + 1, rds))
            else:
                stage_refs[0, :, pl.ds(0, wt)] = jnp.maximum(
                    (p_cw + comm_cw[b, r, :, pl.ds(0, wt)].astype(jnp.float32))
                    * scale, 0.0)
                cp = pltpu.make_async_copy(
                    stage_refs.at[0, slice(None), pl.ds(0, wt)],
                    out_hbm.at[:, pl.ds(col_cw, wt)],
                    copy_sems.at[0])
                cp.start()
                stage_refs[1, :, pl.ds(0, wt)] = jnp.maximum(
                    (p_ccw + comm_ccw[b, r, :, pl.ds(0, wt)].astype(jnp.float32))
                    * scale, 0.0)
                cp2 = pltpu.make_async_copy(
                    stage_refs.at[1, slice(None), pl.ds(0, wt)],
                    out_hbm.at[:, pl.ds(col_ccw, wt)],
                    copy_sems.at[1])
                cp2.start()
                if next_tile < N_TILES:
                    queue.append((next_tile, 0, init_tile(next_tile)))
                    next_tile += 1
                cp.wait()
                cp2.wait()
            if step in hooks:
                run_staging(hooks.pop(step))
            step += 1

    return pl.pallas_call(
        body,
        out_shape=jax.ShapeDtypeStruct((M_BLK, N), jnp.float32),
        in_specs=[
            pl.BlockSpec(memory_space=pl.ANY),
            pl.BlockSpec(memory_space=pl.ANY),
            pl.BlockSpec(memory_space=pltpu.SMEM),
            pl.BlockSpec(memory_space=pltpu.SMEM),
        ],
        out_specs=pl.BlockSpec(memory_space=pl.ANY),
        scratch_shapes=[
            pltpu.VMEM((M, K), jnp.float8_e4m3fn),
            pltpu.VMEM((K, N), jnp.float8_e5m2),
            pltpu.VMEM((4, M_BLK, WT), jnp.float32),
            pltpu.VMEM((2, 3, M_BLK, WT), jnp.bfloat16),
            pltpu.VMEM((2, 3, M_BLK, WT), jnp.bfloat16),
            pltpu.SemaphoreType.DMA((N_TILES, N_HOP)),
            pltpu.SemaphoreType.DMA((N_TILES, N_HOP)),
            pltpu.SemaphoreType.DMA((N_TILES, N_HOP)),
            pltpu.SemaphoreType.DMA((N_TILES, N_HOP)),
            pltpu.SemaphoreType.DMA((4,)),
        ],
        compiler_params=pltpu.CompilerParams(
            collective_id=0,
            vmem_limit_bytes=60 * 1024 * 1024,
        ),
    )(x, w_mat, scale_x, scale_w)
